# Initial kernel scaffold; baseline (speedup 1.0000x reference)
import jax
import jax.numpy as jnp
from jax.experimental import pallas as pl


def kernel(priors, potential, src_nodes, dst_nodes, rev_edges):
    raise NotImplementedError("write your pallas kernel here")


# speedup vs baseline: 23.0316x; 23.0316x over previous
import functools

import jax
import jax.numpy as jnp
from jax import lax
from jax.experimental import pallas as pl
from jax.experimental.pallas import tpu as pltpu
import jax.experimental.pallas.tpu_sc as plsc

MAX_ITERS_K = 10
NT = 16
RW = 128
BR = 16
LN = 4096
DMAX = 16.0
LUT_SCALE = LN / DMAX
PAD_SPREAD = 1024


def _feval(lutb_v, luts_v, delta):
    a = jnp.abs(delta)
    scaled = jnp.minimum(a * jnp.float32(LUT_SCALE), jnp.float32(LN) - 0.5)
    idx = scaled.astype(jnp.int32)
    frac = scaled - idx.astype(jnp.float32)
    g = plsc.load_gather(lutb_v, [idx]) + frac * plsc.load_gather(luts_v, [idx])
    return jnp.where(delta < 0.0, -g, g)


def _make_bp(npad, rows, nblk, interpret=False):
    rpt = nblk * BR
    sl = npad // NT
    mesh = plsc.VectorSubcoreMesh(
        core_axis_name="c", subcore_axis_name="s",
        num_cores=1, num_subcores=NT)

    def body(srcR, dstR, pi_hbm, lutb_hbm, luts_hbm,
             b0_hbm, b1_hbm, MR, WR, beta_hbm,
             beta_tab, lutb_v, luts_v,
             src_blk, dst_blk, m_blk, w_blk, mo_blk, wo_blk,
             pi_sl, acc_sl, beta_sl, zz_sl, prob_sl, acc_shared):
        wid = lax.axis_index("s")
        base_row = wid * rpt
        nbase = wid * sl

        pltpu.sync_copy(lutb_hbm, lutb_v)
        pltpu.sync_copy(luts_hbm, luts_v)
        pltpu.sync_copy(pi_hbm.at[pl.ds(nbase, sl)], pi_sl)

        def _z16(v, _):
            zz_sl[pl.ds(v * 16, 16)] = jnp.zeros((16,), jnp.float32)
        lax.fori_loop(0, sl // 16, _z16, None)

        def _zrow(j, _):
            for c in range(RW // 16):
                mo_blk[j, pl.ds(c * 16, 16)] = jnp.zeros((16,), jnp.float32)
        lax.fori_loop(0, BR, _zrow, None)

        def _zmw(b, _):
            r0 = base_row + b * BR
            pltpu.sync_copy(mo_blk, MR.at[pl.ds(r0, BR)])
            pltpu.sync_copy(mo_blk, WR.at[pl.ds(r0, BR)])
        lax.fori_loop(0, nblk, _zmw, None)

        pltpu.sync_copy(zz_sl, acc_shared.at[pl.ds(nbase, sl)])
        plsc.subcore_barrier()

        def iter_body(t, _):
            @pl.when(t == 0)
            def _():
                pltpu.sync_copy(pi_hbm, beta_tab)

            @pl.when(t != 0)
            def _():
                pltpu.sync_copy(beta_hbm, beta_tab)

            def blk_body(b, _):
                r0 = base_row + b * BR
                pltpu.sync_copy(srcR.at[pl.ds(r0, BR)], src_blk)
                pltpu.sync_copy(dstR.at[pl.ds(r0, BR)], dst_blk)
                pltpu.sync_copy(MR.at[pl.ds(r0, BR)], m_blk)
                pltpu.sync_copy(WR.at[pl.ds(r0, BR)], w_blk)

                def row_body(j, _):
                    for c in range(RW // 16):
                        cs = pl.ds(c * 16, 16)
                        sv = src_blk[j, cs]
                        dv = dst_blk[j, cs]
                        mv = m_blk[j, cs]
                        wv = w_blk[j, cs]
                        bs = plsc.load_gather(beta_tab, [sv])
                        bd = plsc.load_gather(beta_tab, [dv])
                        mo_blk[j, cs] = _feval(lutb_v, luts_v, bs - wv)
                        wo_blk[j, cs] = _feval(lutb_v, luts_v, bd - mv)
                lax.fori_loop(0, BR, row_body, None)

                pltpu.sync_copy(mo_blk, MR.at[pl.ds(r0, BR)])
                pltpu.sync_copy(wo_blk, WR.at[pl.ds(r0, BR)])

                def scat_body(j, _):
                    pltpu.sync_copy(wo_blk.at[j],
                                    acc_shared.at[src_blk.at[j]], add=True)
                lax.fori_loop(0, BR, scat_body, None)
            lax.fori_loop(0, nblk, blk_body, None)
            plsc.subcore_barrier()

            pltpu.sync_copy(acc_shared.at[pl.ds(nbase, sl)], acc_sl)

            def nb(v, _):
                vs = pl.ds(v * 16, 16)
                beta_sl[vs] = pi_sl[vs] + acc_sl[vs]
            lax.fori_loop(0, sl // 16, nb, None)

            pltpu.sync_copy(beta_sl, beta_hbm.at[pl.ds(nbase, sl)])
            pltpu.sync_copy(zz_sl, acc_shared.at[pl.ds(nbase, sl)])

            @pl.when(t == MAX_ITERS_K - 1)
            def _():
                def fin(v, _):
                    vs = pl.ds(v * 16, 16)
                    p0 = 1.0 / (1.0 + jnp.exp(-beta_sl[vs]))
                    prob_sl[vs] = p0
                lax.fori_loop(0, sl // 16, fin, None)
                pltpu.sync_copy(prob_sl, b0_hbm.at[pl.ds(nbase, sl)])

                def fin2(v, _):
                    vs = pl.ds(v * 16, 16)
                    prob_sl[vs] = 1.0 - prob_sl[vs]
                lax.fori_loop(0, sl // 16, fin2, None)
                pltpu.sync_copy(prob_sl, b1_hbm.at[pl.ds(nbase, sl)])

            plsc.subcore_barrier()
        lax.fori_loop(0, MAX_ITERS_K, iter_body, None)

    f32 = jnp.float32
    return pl.kernel(
        body,
        out_type=(
            jax.ShapeDtypeStruct((npad,), f32),
            jax.ShapeDtypeStruct((npad,), f32),
            jax.ShapeDtypeStruct((rows, RW), f32),
            jax.ShapeDtypeStruct((rows, RW), f32),
            jax.ShapeDtypeStruct((npad,), f32),
        ),
        mesh=mesh,
        scratch_types=[
            pltpu.VMEM((npad,), f32),
            pltpu.VMEM((LN,), f32),
            pltpu.VMEM((LN,), f32),
            pltpu.VMEM((BR, RW), jnp.int32),
            pltpu.VMEM((BR, RW), jnp.int32),
            pltpu.VMEM((BR, RW), f32),
            pltpu.VMEM((BR, RW), f32),
            pltpu.VMEM((BR, RW), f32),
            pltpu.VMEM((BR, RW), f32),
            pltpu.VMEM((npad // NT,), f32),
            pltpu.VMEM((npad // NT,), f32),
            pltpu.VMEM((npad // NT,), f32),
            pltpu.VMEM((npad // NT,), f32),
            pltpu.VMEM((npad // NT,), f32),
            pltpu.VMEM_SHARED((npad,), f32),
        ],
        compiler_params=pltpu.CompilerParams(needs_layout_passes=False),
        interpret=interpret,
    )


def _prepare(priors, potential, src_nodes, dst_nodes):
    n = priors.shape[0]
    nnz = src_nodes.shape[0]
    blk = BR * RW
    nblk = -(-nnz // (NT * blk))
    rows = NT * nblk * BR
    nnz_pad = rows * RW
    npad = -(-(n + PAD_SPREAD) // (NT * 16)) * (NT * 16)

    i32 = jnp.int32
    pad_idx = (n + (jnp.arange(nnz_pad - nnz, dtype=i32) % PAD_SPREAD))
    src_p = jnp.concatenate(
        [src_nodes.astype(i32), pad_idx]).reshape(rows, RW)
    dst_p = jnp.concatenate(
        [dst_nodes.astype(i32), pad_idx]).reshape(rows, RW)

    logpr = jnp.log(priors)
    pi = logpr[:, 0] - logpr[:, 1]
    pi_pad = jnp.concatenate(
        [pi, jnp.zeros((npad - n,), jnp.float32)])

    k = potential[0, 0] / potential[0, 1]
    grid = jnp.arange(LN + 1, dtype=jnp.float32) * jnp.float32(1.0 / LUT_SCALE)
    tt = jnp.exp(-grid)
    fv = jnp.log((k + tt) / (1.0 + k * tt))
    lutb = fv[:LN]
    luts = fv[1:] - fv[:-1]
    return npad, rows, nblk, src_p, dst_p, pi_pad, lutb, luts


def kernel(priors, potential, src_nodes, dst_nodes, rev_edges):
    del rev_edges
    n = priors.shape[0]
    npad, rows, nblk, src_p, dst_p, pi_pad, lutb, luts = _prepare(
        priors, potential, src_nodes, dst_nodes)
    fn = _make_bp(npad, rows, nblk)
    b0, b1, _m, _w, _beta = fn(src_p, dst_p, pi_pad, lutb, luts)
    return jnp.stack([b0[:n], b1[:n]], axis=1)


# speedup vs baseline: 33.6958x; 1.4630x over previous
import functools

import jax
import jax.numpy as jnp
from jax import lax
from jax.experimental import pallas as pl
from jax.experimental.pallas import tpu as pltpu
import jax.experimental.pallas.tpu_sc as plsc

MAX_ITERS_K = 10
NT = 16
RW = 128
BR = 16
LN = 4096
DMAX = 16.0
LUT_SCALE = LN / DMAX
PAD_SPREAD = 1024


def _feval(lutb_v, luts_v, delta):
    a = jnp.abs(delta)
    scaled = jnp.minimum(a * jnp.float32(LUT_SCALE), jnp.float32(LN) - 0.5)
    idx = scaled.astype(jnp.int32)
    frac = scaled - idx.astype(jnp.float32)
    g = plsc.load_gather(lutb_v, [idx]) + frac * plsc.load_gather(luts_v, [idx])
    return jnp.where(delta < 0.0, -g, g)


def _make_bp(npad, rows, nblk, interpret=False):
    rpt = nblk * BR
    sl = npad // NT
    mesh = plsc.VectorSubcoreMesh(
        core_axis_name="c", subcore_axis_name="s",
        num_cores=1, num_subcores=NT)

    def body(srcR, dstR, pi_hbm, lutb_hbm, luts_hbm,
             b0_hbm, b1_hbm, MR, WR, beta_hbm,
             beta_tab, lutb_v, luts_v,
             src_blk, dst_blk, m_blk, w_blk, mo_blk, wo_blk,
             pi_sl, acc_sl, beta_sl, zz_sl, prob_sl, acc_shared,
             sem_in, sem_out, sem_scat):
        wid = lax.axis_index("s")
        base_row = wid * rpt
        nbase = wid * sl

        pltpu.sync_copy(lutb_hbm, lutb_v)
        pltpu.sync_copy(luts_hbm, luts_v)
        pltpu.sync_copy(pi_hbm.at[pl.ds(nbase, sl)], pi_sl)

        def _z16(v, _):
            zz_sl[pl.ds(v * 16, 16)] = jnp.zeros((16,), jnp.float32)
        lax.fori_loop(0, sl // 16, _z16, None)

        def _zrow(j, _):
            for c in range(RW // 16):
                mo_blk[j, pl.ds(c * 16, 16)] = jnp.zeros((16,), jnp.float32)
        lax.fori_loop(0, BR, _zrow, None)

        def _zmw(b, _):
            r0 = base_row + b * BR
            pltpu.sync_copy(mo_blk, MR.at[pl.ds(r0, BR)])
            pltpu.sync_copy(mo_blk, WR.at[pl.ds(r0, BR)])
        lax.fori_loop(0, nblk, _zmw, None)

        pltpu.sync_copy(zz_sl, acc_shared.at[pl.ds(nbase, sl)])
        plsc.subcore_barrier()

        def iter_body(t, _):
            @pl.when(t == 0)
            def _():
                pltpu.sync_copy(pi_hbm, beta_tab)

            @pl.when(t != 0)
            def _():
                pltpu.sync_copy(beta_hbm, beta_tab)

            def blk_body(b, _):
                r0 = base_row + b * BR
                ins = [
                    pltpu.async_copy(srcR.at[pl.ds(r0, BR)], src_blk, sem_in),
                    pltpu.async_copy(dstR.at[pl.ds(r0, BR)], dst_blk, sem_in),
                    pltpu.async_copy(MR.at[pl.ds(r0, BR)], m_blk, sem_in),
                    pltpu.async_copy(WR.at[pl.ds(r0, BR)], w_blk, sem_in),
                ]
                for dsc in ins:
                    dsc.wait()

                def row_body(j, _):
                    for c in range(RW // 16):
                        cs = pl.ds(c * 16, 16)
                        sv = src_blk[j, cs]
                        dv = dst_blk[j, cs]
                        mv = m_blk[j, cs]
                        wv = w_blk[j, cs]
                        bs = plsc.load_gather(beta_tab, [sv])
                        bd = plsc.load_gather(beta_tab, [dv])
                        mo_blk[j, cs] = _feval(lutb_v, luts_v, bs - wv)
                        wo_blk[j, cs] = _feval(lutb_v, luts_v, bd - mv)
                lax.fori_loop(0, BR, row_body, None)

                outs = [
                    pltpu.async_copy(mo_blk, MR.at[pl.ds(r0, BR)], sem_out),
                    pltpu.async_copy(wo_blk, WR.at[pl.ds(r0, BR)], sem_out),
                ]
                scats = [
                    pltpu.async_copy(wo_blk.at[j],
                                     acc_shared.at[src_blk.at[j]],
                                     sem_scat, add=True)
                    for j in range(BR)
                ]
                for dsc in outs:
                    dsc.wait()
                for dsc in scats:
                    dsc.wait()
            lax.fori_loop(0, nblk, blk_body, None)
            plsc.subcore_barrier()

            pltpu.sync_copy(acc_shared.at[pl.ds(nbase, sl)], acc_sl)

            def nb(v, _):
                vs = pl.ds(v * 16, 16)
                beta_sl[vs] = pi_sl[vs] + acc_sl[vs]
            lax.fori_loop(0, sl // 16, nb, None)

            pltpu.sync_copy(beta_sl, beta_hbm.at[pl.ds(nbase, sl)])
            pltpu.sync_copy(zz_sl, acc_shared.at[pl.ds(nbase, sl)])

            @pl.when(t == MAX_ITERS_K - 1)
            def _():
                def fin(v, _):
                    vs = pl.ds(v * 16, 16)
                    p0 = 1.0 / (1.0 + jnp.exp(-beta_sl[vs]))
                    prob_sl[vs] = p0
                lax.fori_loop(0, sl // 16, fin, None)
                pltpu.sync_copy(prob_sl, b0_hbm.at[pl.ds(nbase, sl)])

                def fin2(v, _):
                    vs = pl.ds(v * 16, 16)
                    prob_sl[vs] = 1.0 - prob_sl[vs]
                lax.fori_loop(0, sl // 16, fin2, None)
                pltpu.sync_copy(prob_sl, b1_hbm.at[pl.ds(nbase, sl)])

            plsc.subcore_barrier()
        lax.fori_loop(0, MAX_ITERS_K, iter_body, None)

    f32 = jnp.float32
    return pl.kernel(
        body,
        out_type=(
            jax.ShapeDtypeStruct((npad,), f32),
            jax.ShapeDtypeStruct((npad,), f32),
            jax.ShapeDtypeStruct((rows, RW), f32),
            jax.ShapeDtypeStruct((rows, RW), f32),
            jax.ShapeDtypeStruct((npad,), f32),
        ),
        mesh=mesh,
        scratch_types=[
            pltpu.VMEM((npad,), f32),
            pltpu.VMEM((LN,), f32),
            pltpu.VMEM((LN,), f32),
            pltpu.VMEM((BR, RW), jnp.int32),
            pltpu.VMEM((BR, RW), jnp.int32),
            pltpu.VMEM((BR, RW), f32),
            pltpu.VMEM((BR, RW), f32),
            pltpu.VMEM((BR, RW), f32),
            pltpu.VMEM((BR, RW), f32),
            pltpu.VMEM((npad // NT,), f32),
            pltpu.VMEM((npad // NT,), f32),
            pltpu.VMEM((npad // NT,), f32),
            pltpu.VMEM((npad // NT,), f32),
            pltpu.VMEM((npad // NT,), f32),
            pltpu.VMEM_SHARED((npad,), f32),
            pltpu.SemaphoreType.DMA,
            pltpu.SemaphoreType.DMA,
            pltpu.SemaphoreType.DMA,
        ],
        compiler_params=pltpu.CompilerParams(needs_layout_passes=False),
        interpret=interpret,
    )


def _prepare(priors, potential, src_nodes, dst_nodes):
    n = priors.shape[0]
    nnz = src_nodes.shape[0]
    blk = BR * RW
    nblk = -(-nnz // (NT * blk))
    rows = NT * nblk * BR
    nnz_pad = rows * RW
    npad = -(-(n + PAD_SPREAD) // (NT * 16)) * (NT * 16)

    i32 = jnp.int32
    pad_idx = (n + (jnp.arange(nnz_pad - nnz, dtype=i32) % PAD_SPREAD))
    src_p = jnp.concatenate(
        [src_nodes.astype(i32), pad_idx]).reshape(rows, RW)
    dst_p = jnp.concatenate(
        [dst_nodes.astype(i32), pad_idx]).reshape(rows, RW)

    logpr = jnp.log(priors)
    pi = logpr[:, 0] - logpr[:, 1]
    pi_pad = jnp.concatenate(
        [pi, jnp.zeros((npad - n,), jnp.float32)])

    k = potential[0, 0] / potential[0, 1]
    grid = jnp.arange(LN + 1, dtype=jnp.float32) * jnp.float32(1.0 / LUT_SCALE)
    tt = jnp.exp(-grid)
    fv = jnp.log((k + tt) / (1.0 + k * tt))
    lutb = fv[:LN]
    luts = fv[1:] - fv[:-1]
    return npad, rows, nblk, src_p, dst_p, pi_pad, lutb, luts


def kernel(priors, potential, src_nodes, dst_nodes, rev_edges):
    del rev_edges
    n = priors.shape[0]
    npad, rows, nblk, src_p, dst_p, pi_pad, lutb, luts = _prepare(
        priors, potential, src_nodes, dst_nodes)
    fn = _make_bp(npad, rows, nblk)
    b0, b1, _m, _w, _beta = fn(src_p, dst_p, pi_pad, lutb, luts)
    return jnp.stack([b0[:n], b1[:n]], axis=1)


# speedup vs baseline: 34.3317x; 1.0189x over previous
import jax
import jax.numpy as jnp
from jax import lax
from jax.experimental import pallas as pl
from jax.experimental.pallas import tpu as pltpu
import jax.experimental.pallas.tpu_sc as plsc

MAX_ITERS_K = 10
NT = 16
BLK = 2048
LN = 4096
DMAX = 16.0
LUT_SCALE = LN / DMAX
PAD_SPREAD = 1024


def _feval(lutb_v, luts_v, delta):
    a = jnp.abs(delta)
    scaled = jnp.minimum(a * jnp.float32(LUT_SCALE), jnp.float32(LN) - 0.5)
    idx = scaled.astype(jnp.int32)
    frac = scaled - idx.astype(jnp.float32)
    g = plsc.load_gather(lutb_v, [idx]) + frac * plsc.load_gather(luts_v, [idx])
    return jnp.where(delta < 0.0, -g, g)


def _make_bp(npad, nnz_pad, nblk, interpret=False):
    ept = nblk * BLK
    sl = npad // NT
    mesh = plsc.VectorSubcoreMesh(
        core_axis_name="c", subcore_axis_name="s",
        num_cores=1, num_subcores=NT)

    def body(srcE, dstE, pi_hbm, lutb_hbm, luts_hbm,
             b0_hbm, b1_hbm, ME, WE, beta_hbm,
             beta_tab, lutb_v, luts_v,
             src_blk, dst_blk, m_blk, w_blk, mo_blk, wo_blk,
             pi_sl, acc_sl, beta_sl, zz_sl, prob_sl, acc_shared,
             sem_in, sem_out, sem_scat):
        wid = lax.axis_index("s")
        ebase = wid * ept
        nbase = wid * sl

        pltpu.sync_copy(lutb_hbm, lutb_v)
        pltpu.sync_copy(luts_hbm, luts_v)
        pltpu.sync_copy(pi_hbm.at[pl.ds(nbase, sl)], pi_sl)

        def _z16(v, _):
            zz_sl[pl.ds(v * 16, 16)] = jnp.zeros((16,), jnp.float32)
        lax.fori_loop(0, sl // 16, _z16, None)

        def _zb(v, _):
            mo_blk[pl.ds(v * 16, 16)] = jnp.zeros((16,), jnp.float32)
        lax.fori_loop(0, BLK // 16, _zb, None)

        def _zmw(b, _):
            e0 = ebase + b * BLK
            pltpu.sync_copy(mo_blk, ME.at[pl.ds(e0, BLK)])
            pltpu.sync_copy(mo_blk, WE.at[pl.ds(e0, BLK)])
        lax.fori_loop(0, nblk, _zmw, None)

        pltpu.sync_copy(zz_sl, acc_shared.at[pl.ds(nbase, sl)])
        plsc.subcore_barrier()

        def iter_body(t, _):
            @pl.when(t == 0)
            def _():
                pltpu.sync_copy(pi_hbm, beta_tab)

            @pl.when(t != 0)
            def _():
                pltpu.sync_copy(beta_hbm, beta_tab)

            def blk_body(b, _):
                e0 = ebase + b * BLK
                ins = [
                    pltpu.async_copy(srcE.at[pl.ds(e0, BLK)], src_blk, sem_in),
                    pltpu.async_copy(dstE.at[pl.ds(e0, BLK)], dst_blk, sem_in),
                    pltpu.async_copy(ME.at[pl.ds(e0, BLK)], m_blk, sem_in),
                    pltpu.async_copy(WE.at[pl.ds(e0, BLK)], w_blk, sem_in),
                ]
                for dsc in ins:
                    dsc.wait()

                def vec_body(v, _):
                    cs = pl.ds(v * 16, 16)
                    sv = src_blk[cs]
                    dv = dst_blk[cs]
                    mv = m_blk[cs]
                    wv = w_blk[cs]
                    bs = plsc.load_gather(beta_tab, [sv])
                    bd = plsc.load_gather(beta_tab, [dv])
                    mo_blk[cs] = _feval(lutb_v, luts_v, bs - wv)
                    wo_blk[cs] = _feval(lutb_v, luts_v, bd - mv)
                lax.fori_loop(0, BLK // 16, vec_body, None)

                outs = [
                    pltpu.async_copy(mo_blk, ME.at[pl.ds(e0, BLK)], sem_out),
                    pltpu.async_copy(wo_blk, WE.at[pl.ds(e0, BLK)], sem_out),
                ]
                scat = pltpu.async_copy(wo_blk, acc_shared.at[src_blk],
                                        sem_scat, add=True)
                for dsc in outs:
                    dsc.wait()
                scat.wait()
            lax.fori_loop(0, nblk, blk_body, None)
            plsc.subcore_barrier()

            pltpu.sync_copy(acc_shared.at[pl.ds(nbase, sl)], acc_sl)

            def nb(v, _):
                vs = pl.ds(v * 16, 16)
                beta_sl[vs] = pi_sl[vs] + acc_sl[vs]
            lax.fori_loop(0, sl // 16, nb, None)

            pltpu.sync_copy(beta_sl, beta_hbm.at[pl.ds(nbase, sl)])
            pltpu.sync_copy(zz_sl, acc_shared.at[pl.ds(nbase, sl)])

            @pl.when(t == MAX_ITERS_K - 1)
            def _():
                def fin(v, _):
                    vs = pl.ds(v * 16, 16)
                    p0 = 1.0 / (1.0 + jnp.exp(-beta_sl[vs]))
                    prob_sl[vs] = p0
                lax.fori_loop(0, sl // 16, fin, None)
                pltpu.sync_copy(prob_sl, b0_hbm.at[pl.ds(nbase, sl)])

                def fin2(v, _):
                    vs = pl.ds(v * 16, 16)
                    prob_sl[vs] = 1.0 - prob_sl[vs]
                lax.fori_loop(0, sl // 16, fin2, None)
                pltpu.sync_copy(prob_sl, b1_hbm.at[pl.ds(nbase, sl)])

            plsc.subcore_barrier()
        lax.fori_loop(0, MAX_ITERS_K, iter_body, None)

    f32 = jnp.float32
    return pl.kernel(
        body,
        out_type=(
            jax.ShapeDtypeStruct((npad,), f32),
            jax.ShapeDtypeStruct((npad,), f32),
            jax.ShapeDtypeStruct((nnz_pad,), f32),
            jax.ShapeDtypeStruct((nnz_pad,), f32),
            jax.ShapeDtypeStruct((npad,), f32),
        ),
        mesh=mesh,
        scratch_types=[
            pltpu.VMEM((npad,), f32),
            pltpu.VMEM((LN,), f32),
            pltpu.VMEM((LN,), f32),
            pltpu.VMEM((BLK,), jnp.int32),
            pltpu.VMEM((BLK,), jnp.int32),
            pltpu.VMEM((BLK,), f32),
            pltpu.VMEM((BLK,), f32),
            pltpu.VMEM((BLK,), f32),
            pltpu.VMEM((BLK,), f32),
            pltpu.VMEM((npad // NT,), f32),
            pltpu.VMEM((npad // NT,), f32),
            pltpu.VMEM((npad // NT,), f32),
            pltpu.VMEM((npad // NT,), f32),
            pltpu.VMEM((npad // NT,), f32),
            pltpu.VMEM_SHARED((npad,), f32),
            pltpu.SemaphoreType.DMA,
            pltpu.SemaphoreType.DMA,
            pltpu.SemaphoreType.DMA,
        ],
        compiler_params=pltpu.CompilerParams(needs_layout_passes=False),
        interpret=interpret,
    )


def _prepare(priors, potential, src_nodes, dst_nodes):
    n = priors.shape[0]
    nnz = src_nodes.shape[0]
    nblk = -(-nnz // (NT * BLK))
    nnz_pad = NT * nblk * BLK
    npad = -(-(n + PAD_SPREAD) // (NT * 16)) * (NT * 16)

    i32 = jnp.int32
    pad_idx = (n + (jnp.arange(nnz_pad - nnz, dtype=i32) % PAD_SPREAD))
    src_p = jnp.concatenate([src_nodes.astype(i32), pad_idx])
    dst_p = jnp.concatenate([dst_nodes.astype(i32), pad_idx])

    logpr = jnp.log(priors)
    pi = logpr[:, 0] - logpr[:, 1]
    pi_pad = jnp.concatenate(
        [pi, jnp.zeros((npad - n,), jnp.float32)])

    k = potential[0, 0] / potential[0, 1]
    grid = jnp.arange(LN + 1, dtype=jnp.float32) * jnp.float32(1.0 / LUT_SCALE)
    tt = jnp.exp(-grid)
    fv = jnp.log((k + tt) / (1.0 + k * tt))
    lutb = fv[:LN]
    luts = fv[1:] - fv[:-1]
    return npad, nnz_pad, nblk, src_p, dst_p, pi_pad, lutb, luts


def kernel(priors, potential, src_nodes, dst_nodes, rev_edges):
    del rev_edges
    n = priors.shape[0]
    npad, nnz_pad, nblk, src_p, dst_p, pi_pad, lutb, luts = _prepare(
        priors, potential, src_nodes, dst_nodes)
    fn = _make_bp(npad, nnz_pad, nblk)
    b0, b1, _m, _w, _beta = fn(src_p, dst_p, pi_pad, lutb, luts)
    return jnp.stack([b0[:n], b1[:n]], axis=1)


# speedup vs baseline: 66.2719x; 1.9303x over previous
import jax
import jax.numpy as jnp
from jax import lax
from jax.experimental import pallas as pl
from jax.experimental.pallas import tpu as pltpu
import jax.experimental.pallas.tpu_sc as plsc

MAX_ITERS_K = 10
NT = 16
BLK = 2048
LN = 4096
DMAX = 16.0
LUT_SCALE = LN / DMAX
PAD_SPREAD = 1024


def _feval(lutb_v, luts_v, delta):
    a = jnp.abs(delta)
    scaled = jnp.minimum(a * jnp.float32(LUT_SCALE), jnp.float32(LN) - 0.5)
    idx = scaled.astype(jnp.int32)
    frac = scaled - idx.astype(jnp.float32)
    g = plsc.load_gather(lutb_v, [idx]) + frac * plsc.load_gather(luts_v, [idx])
    return jnp.where(delta < 0.0, -g, g)


def _make_bp(npad, nnz_pad, nblk, interpret=False):
    ept = nblk * BLK
    sl = npad // NT
    mesh = plsc.VectorSubcoreMesh(
        core_axis_name="c", subcore_axis_name="s",
        num_cores=1, num_subcores=NT)

    def body(srcE, dstE, pi_hbm, lutb_hbm, luts_hbm,
             b0_hbm, b1_hbm, ME, WE, beta_hbm,
             beta_tab, lutb_v, luts_v,
             src_blk, dst_blk, m_blk, w_blk, mo_blk, wo_blk,
             pi_sl, acc_sl, beta_sl, zz_sl, prob_sl, acc_shared,
             sem_in, sem_out, sem_scat):
        wid = lax.axis_index("s")
        ebase = wid * ept
        nbase = wid * sl

        pltpu.sync_copy(lutb_hbm, lutb_v)
        pltpu.sync_copy(luts_hbm, luts_v)
        pltpu.sync_copy(pi_hbm.at[pl.ds(nbase, sl)], pi_sl)

        def _z16(v, _):
            zz_sl[pl.ds(v * 16, 16)] = jnp.zeros((16,), jnp.float32)
        lax.fori_loop(0, sl // 16, _z16, None)

        def _zb(v, _):
            mo_blk[pl.ds(v * 16, 16)] = jnp.zeros((16,), jnp.float32)
        lax.fori_loop(0, BLK // 16, _zb, None)

        def _zmw(b, _):
            e0 = ebase + b * BLK
            pltpu.sync_copy(mo_blk, ME.at[pl.ds(e0, BLK)])
            pltpu.sync_copy(mo_blk, WE.at[pl.ds(e0, BLK)])
        lax.fori_loop(0, nblk, _zmw, None)

        pltpu.sync_copy(zz_sl, acc_shared.at[pl.ds(nbase, sl)])
        plsc.subcore_barrier()

        def iter_body(t, _):
            @pl.when(t == 0)
            def _():
                pltpu.sync_copy(pi_hbm, beta_tab)

            @pl.when(t != 0)
            def _():
                pltpu.sync_copy(beta_hbm, beta_tab)

            def blk_body(b, _):
                e0 = ebase + b * BLK
                ins = [
                    pltpu.async_copy(srcE.at[pl.ds(e0, BLK)], src_blk, sem_in),
                    pltpu.async_copy(dstE.at[pl.ds(e0, BLK)], dst_blk, sem_in),
                    pltpu.async_copy(ME.at[pl.ds(e0, BLK)], m_blk, sem_in),
                    pltpu.async_copy(WE.at[pl.ds(e0, BLK)], w_blk, sem_in),
                ]
                for dsc in ins:
                    dsc.wait()

                @plsc.parallel_loop(0, BLK // 16, 1, unroll=4)
                def vec_body(v):
                    cs = pl.ds(v * 16, 16)
                    sv = src_blk[cs]
                    dv = dst_blk[cs]
                    mv = m_blk[cs]
                    wv = w_blk[cs]
                    bs = plsc.load_gather(beta_tab, [sv])
                    bd = plsc.load_gather(beta_tab, [dv])
                    mo_blk[cs] = _feval(lutb_v, luts_v, bs - wv)
                    wo_blk[cs] = _feval(lutb_v, luts_v, bd - mv)

                outs = [
                    pltpu.async_copy(mo_blk, ME.at[pl.ds(e0, BLK)], sem_out),
                    pltpu.async_copy(wo_blk, WE.at[pl.ds(e0, BLK)], sem_out),
                ]
                scat = pltpu.async_copy(wo_blk, acc_shared.at[src_blk],
                                        sem_scat, add=True)
                for dsc in outs:
                    dsc.wait()
                scat.wait()
            lax.fori_loop(0, nblk, blk_body, None)
            plsc.subcore_barrier()

            pltpu.sync_copy(acc_shared.at[pl.ds(nbase, sl)], acc_sl)

            def nb(v, _):
                vs = pl.ds(v * 16, 16)
                beta_sl[vs] = pi_sl[vs] + acc_sl[vs]
            lax.fori_loop(0, sl // 16, nb, None)

            pltpu.sync_copy(beta_sl, beta_hbm.at[pl.ds(nbase, sl)])
            pltpu.sync_copy(zz_sl, acc_shared.at[pl.ds(nbase, sl)])

            @pl.when(t == MAX_ITERS_K - 1)
            def _():
                def fin(v, _):
                    vs = pl.ds(v * 16, 16)
                    p0 = 1.0 / (1.0 + jnp.exp(-beta_sl[vs]))
                    prob_sl[vs] = p0
                lax.fori_loop(0, sl // 16, fin, None)
                pltpu.sync_copy(prob_sl, b0_hbm.at[pl.ds(nbase, sl)])

                def fin2(v, _):
                    vs = pl.ds(v * 16, 16)
                    prob_sl[vs] = 1.0 - prob_sl[vs]
                lax.fori_loop(0, sl // 16, fin2, None)
                pltpu.sync_copy(prob_sl, b1_hbm.at[pl.ds(nbase, sl)])

            plsc.subcore_barrier()
        lax.fori_loop(0, MAX_ITERS_K, iter_body, None)

    f32 = jnp.float32
    return pl.kernel(
        body,
        out_type=(
            jax.ShapeDtypeStruct((npad,), f32),
            jax.ShapeDtypeStruct((npad,), f32),
            jax.ShapeDtypeStruct((nnz_pad,), f32),
            jax.ShapeDtypeStruct((nnz_pad,), f32),
            jax.ShapeDtypeStruct((npad,), f32),
        ),
        mesh=mesh,
        scratch_types=[
            pltpu.VMEM((npad,), f32),
            pltpu.VMEM((LN,), f32),
            pltpu.VMEM((LN,), f32),
            pltpu.VMEM((BLK,), jnp.int32),
            pltpu.VMEM((BLK,), jnp.int32),
            pltpu.VMEM((BLK,), f32),
            pltpu.VMEM((BLK,), f32),
            pltpu.VMEM((BLK,), f32),
            pltpu.VMEM((BLK,), f32),
            pltpu.VMEM((npad // NT,), f32),
            pltpu.VMEM((npad // NT,), f32),
            pltpu.VMEM((npad // NT,), f32),
            pltpu.VMEM((npad // NT,), f32),
            pltpu.VMEM((npad // NT,), f32),
            pltpu.VMEM_SHARED((npad,), f32),
            pltpu.SemaphoreType.DMA,
            pltpu.SemaphoreType.DMA,
            pltpu.SemaphoreType.DMA,
        ],
        compiler_params=pltpu.CompilerParams(needs_layout_passes=False),
        interpret=interpret,
    )


def _prepare(priors, potential, src_nodes, dst_nodes):
    n = priors.shape[0]
    nnz = src_nodes.shape[0]
    nblk = -(-nnz // (NT * BLK))
    nnz_pad = NT * nblk * BLK
    npad = -(-(n + PAD_SPREAD) // (NT * 16)) * (NT * 16)

    i32 = jnp.int32
    pad_idx = (n + (jnp.arange(nnz_pad - nnz, dtype=i32) % PAD_SPREAD))
    src_p = jnp.concatenate([src_nodes.astype(i32), pad_idx])
    dst_p = jnp.concatenate([dst_nodes.astype(i32), pad_idx])

    logpr = jnp.log(priors)
    pi = logpr[:, 0] - logpr[:, 1]
    pi_pad = jnp.concatenate(
        [pi, jnp.zeros((npad - n,), jnp.float32)])

    k = potential[0, 0] / potential[0, 1]
    grid = jnp.arange(LN + 1, dtype=jnp.float32) * jnp.float32(1.0 / LUT_SCALE)
    tt = jnp.exp(-grid)
    fv = jnp.log((k + tt) / (1.0 + k * tt))
    lutb = fv[:LN]
    luts = fv[1:] - fv[:-1]
    return npad, nnz_pad, nblk, src_p, dst_p, pi_pad, lutb, luts


def kernel(priors, potential, src_nodes, dst_nodes, rev_edges):
    del rev_edges
    n = priors.shape[0]
    npad, nnz_pad, nblk, src_p, dst_p, pi_pad, lutb, luts = _prepare(
        priors, potential, src_nodes, dst_nodes)
    fn = _make_bp(npad, nnz_pad, nblk)
    b0, b1, _m, _w, _beta = fn(src_p, dst_p, pi_pad, lutb, luts)
    return jnp.stack([b0[:n], b1[:n]], axis=1)


# speedup vs baseline: 66.7539x; 1.0073x over previous
import jax
import jax.numpy as jnp
from jax import lax
from jax.experimental import pallas as pl
from jax.experimental.pallas import tpu as pltpu
import jax.experimental.pallas.tpu_sc as plsc

MAX_ITERS_K = 10
NT = 16
BLK = 2048
LN = 4096
DMAX = 16.0
LUT_SCALE = LN / DMAX
PAD_SPREAD = 1024


def _feval(lutb_v, luts_v, delta):
    a = jnp.abs(delta)
    scaled = jnp.minimum(a * jnp.float32(LUT_SCALE), jnp.float32(LN) - 0.5)
    idx = scaled.astype(jnp.int32)
    frac = scaled - idx.astype(jnp.float32)
    g = plsc.load_gather(lutb_v, [idx]) + frac * plsc.load_gather(luts_v, [idx])
    return jnp.where(delta < 0.0, -g, g)


def _make_bp(npad, nnz_pad, nblk, interpret=False):
    ept = nblk * BLK
    sl = npad // NT
    mesh = plsc.VectorSubcoreMesh(
        core_axis_name="c", subcore_axis_name="s",
        num_cores=1, num_subcores=NT)

    def body(srcE, dstE, pi_hbm, lutb_hbm, luts_hbm,
             b0_hbm, b1_hbm, ME, WE, beta_hbm,
             beta_tab, lutb_v, luts_v,
             src_blk, dst_blk, m_blk, w_blk, mo_blk, wo_blk,
             pi_sl, acc_sl, beta_sl, zz_sl, prob_sl, acc_shared,
             sem_in, sem_out, sem_scat):
        wid = lax.axis_index("s")
        ebase = wid * ept
        nbase = wid * sl

        pltpu.sync_copy(lutb_hbm, lutb_v)
        pltpu.sync_copy(luts_hbm, luts_v)
        pltpu.sync_copy(pi_hbm.at[pl.ds(nbase, sl)], pi_sl)

        def _z16(v, _):
            zz_sl[pl.ds(v * 16, 16)] = jnp.zeros((16,), jnp.float32)
        lax.fori_loop(0, sl // 16, _z16, None)

        def _zb(v, _):
            mo_blk[pl.ds(v * 16, 16)] = jnp.zeros((16,), jnp.float32)
        lax.fori_loop(0, BLK // 16, _zb, None)

        def _zmw(b, _):
            e0 = ebase + b * BLK
            pltpu.sync_copy(mo_blk, ME.at[pl.ds(e0, BLK)])
            pltpu.sync_copy(mo_blk, WE.at[pl.ds(e0, BLK)])
        lax.fori_loop(0, nblk, _zmw, None)

        pltpu.sync_copy(zz_sl, acc_shared.at[pl.ds(nbase, sl)])
        plsc.subcore_barrier()

        def iter_body(t, _):
            @pl.when(t == 0)
            def _():
                pltpu.sync_copy(pi_hbm, beta_tab)

            @pl.when(t != 0)
            def _():
                pltpu.sync_copy(beta_hbm, beta_tab)

            def blk_body(b, _):
                e0 = ebase + b * BLK
                ins = [
                    pltpu.async_copy(srcE.at[pl.ds(e0, BLK)], src_blk, sem_in),
                    pltpu.async_copy(dstE.at[pl.ds(e0, BLK)], dst_blk, sem_in),
                    pltpu.async_copy(ME.at[pl.ds(e0, BLK)], m_blk, sem_in),
                    pltpu.async_copy(WE.at[pl.ds(e0, BLK)], w_blk, sem_in),
                ]
                for dsc in ins:
                    dsc.wait()

                @plsc.parallel_loop(0, BLK // 16, 1, unroll=8)
                def vec_body(v):
                    cs = pl.ds(v * 16, 16)
                    sv = src_blk[cs]
                    dv = dst_blk[cs]
                    mv = m_blk[cs]
                    wv = w_blk[cs]
                    bs = plsc.load_gather(beta_tab, [sv])
                    bd = plsc.load_gather(beta_tab, [dv])
                    mo_blk[cs] = _feval(lutb_v, luts_v, bs - wv)
                    wo_blk[cs] = _feval(lutb_v, luts_v, bd - mv)

                outs = [
                    pltpu.async_copy(mo_blk, ME.at[pl.ds(e0, BLK)], sem_out),
                    pltpu.async_copy(wo_blk, WE.at[pl.ds(e0, BLK)], sem_out),
                ]
                scat = pltpu.async_copy(wo_blk, acc_shared.at[src_blk],
                                        sem_scat, add=True)
                for dsc in outs:
                    dsc.wait()
                scat.wait()
            lax.fori_loop(0, nblk, blk_body, None)
            plsc.subcore_barrier()

            pltpu.sync_copy(acc_shared.at[pl.ds(nbase, sl)], acc_sl)

            def nb(v, _):
                vs = pl.ds(v * 16, 16)
                beta_sl[vs] = pi_sl[vs] + acc_sl[vs]
            lax.fori_loop(0, sl // 16, nb, None)

            pltpu.sync_copy(beta_sl, beta_hbm.at[pl.ds(nbase, sl)])
            pltpu.sync_copy(zz_sl, acc_shared.at[pl.ds(nbase, sl)])

            @pl.when(t == MAX_ITERS_K - 1)
            def _():
                def fin(v, _):
                    vs = pl.ds(v * 16, 16)
                    p0 = 1.0 / (1.0 + jnp.exp(-beta_sl[vs]))
                    prob_sl[vs] = p0
                lax.fori_loop(0, sl // 16, fin, None)
                pltpu.sync_copy(prob_sl, b0_hbm.at[pl.ds(nbase, sl)])

                def fin2(v, _):
                    vs = pl.ds(v * 16, 16)
                    prob_sl[vs] = 1.0 - prob_sl[vs]
                lax.fori_loop(0, sl // 16, fin2, None)
                pltpu.sync_copy(prob_sl, b1_hbm.at[pl.ds(nbase, sl)])

            plsc.subcore_barrier()
        lax.fori_loop(0, MAX_ITERS_K, iter_body, None)

    f32 = jnp.float32
    return pl.kernel(
        body,
        out_type=(
            jax.ShapeDtypeStruct((npad,), f32),
            jax.ShapeDtypeStruct((npad,), f32),
            jax.ShapeDtypeStruct((nnz_pad,), f32),
            jax.ShapeDtypeStruct((nnz_pad,), f32),
            jax.ShapeDtypeStruct((npad,), f32),
        ),
        mesh=mesh,
        scratch_types=[
            pltpu.VMEM((npad,), f32),
            pltpu.VMEM((LN,), f32),
            pltpu.VMEM((LN,), f32),
            pltpu.VMEM((BLK,), jnp.int32),
            pltpu.VMEM((BLK,), jnp.int32),
            pltpu.VMEM((BLK,), f32),
            pltpu.VMEM((BLK,), f32),
            pltpu.VMEM((BLK,), f32),
            pltpu.VMEM((BLK,), f32),
            pltpu.VMEM((npad // NT,), f32),
            pltpu.VMEM((npad // NT,), f32),
            pltpu.VMEM((npad // NT,), f32),
            pltpu.VMEM((npad // NT,), f32),
            pltpu.VMEM((npad // NT,), f32),
            pltpu.VMEM_SHARED((npad,), f32),
            pltpu.SemaphoreType.DMA,
            pltpu.SemaphoreType.DMA,
            pltpu.SemaphoreType.DMA,
        ],
        compiler_params=pltpu.CompilerParams(needs_layout_passes=False),
        interpret=interpret,
    )


def _prepare(priors, potential, src_nodes, dst_nodes):
    n = priors.shape[0]
    nnz = src_nodes.shape[0]
    nblk = -(-nnz // (NT * BLK))
    nnz_pad = NT * nblk * BLK
    npad = -(-(n + PAD_SPREAD) // (NT * 16)) * (NT * 16)

    i32 = jnp.int32
    pad_idx = (n + (jnp.arange(nnz_pad - nnz, dtype=i32) % PAD_SPREAD))
    src_p = jnp.concatenate([src_nodes.astype(i32), pad_idx])
    dst_p = jnp.concatenate([dst_nodes.astype(i32), pad_idx])

    logpr = jnp.log(priors)
    pi = logpr[:, 0] - logpr[:, 1]
    pi_pad = jnp.concatenate(
        [pi, jnp.zeros((npad - n,), jnp.float32)])

    k = potential[0, 0] / potential[0, 1]
    grid = jnp.arange(LN + 1, dtype=jnp.float32) * jnp.float32(1.0 / LUT_SCALE)
    tt = jnp.exp(-grid)
    fv = jnp.log((k + tt) / (1.0 + k * tt))
    lutb = fv[:LN]
    luts = fv[1:] - fv[:-1]
    return npad, nnz_pad, nblk, src_p, dst_p, pi_pad, lutb, luts


def kernel(priors, potential, src_nodes, dst_nodes, rev_edges):
    del rev_edges
    n = priors.shape[0]
    npad, nnz_pad, nblk, src_p, dst_p, pi_pad, lutb, luts = _prepare(
        priors, potential, src_nodes, dst_nodes)
    fn = _make_bp(npad, nnz_pad, nblk)
    b0, b1, _m, _w, _beta = fn(src_p, dst_p, pi_pad, lutb, luts)
    return jnp.stack([b0[:n], b1[:n]], axis=1)


# speedup vs baseline: 103.1231x; 1.5448x over previous
import jax
import jax.numpy as jnp
from jax import lax
from jax.experimental import pallas as pl
from jax.experimental.pallas import tpu as pltpu
import jax.experimental.pallas.tpu_sc as plsc

MAX_ITERS_K = 10
NC = 2
NT = 16
BLK = 2048
LN = 4096
DMAX = 16.0
LUT_SCALE = LN / DMAX
PAD_SPREAD = 1024


def _feval(lutb_v, luts_v, delta):
    a = jnp.abs(delta)
    scaled = jnp.minimum(a * jnp.float32(LUT_SCALE), jnp.float32(LN) - 0.5)
    idx = scaled.astype(jnp.int32)
    frac = scaled - idx.astype(jnp.float32)
    g = plsc.load_gather(lutb_v, [idx]) + frac * plsc.load_gather(luts_v, [idx])
    return jnp.where(delta < 0.0, -g, g)


def _make_bp(npad, nnz_pad, nblk, interpret=False):
    ept = nblk * BLK
    slc = npad // NT
    slg = npad // (NC * NT)
    mesh = plsc.VectorSubcoreMesh(
        core_axis_name="c", subcore_axis_name="s",
        num_cores=NC, num_subcores=NT)

    def body(srcE, dstE, pi_hbm, lutb_hbm, luts_hbm,
             b0_hbm, b1_hbm, ME, WE, beta_hbm, acc0H, acc1H,
             beta_tab, lutb_v, luts_v,
             src_blk, dst_blk, m_blk, w_blk, mo_blk, wo_blk,
             pi_sl, acc_sl, acc1_sl, beta_sl, zz_sl, prob_sl, acc_shared,
             sem_in, sem_out, sem_scat, xsem):
        cid = lax.axis_index("c")
        wid = lax.axis_index("s")
        gid = cid * NT + wid
        ebase = gid * ept
        nbase = wid * slc
        gbase = gid * slg

        pltpu.sync_copy(lutb_hbm, lutb_v)
        pltpu.sync_copy(luts_hbm, luts_v)
        pltpu.sync_copy(pi_hbm.at[pl.ds(gbase, slg)], pi_sl)

        def _z16(v, _):
            zz_sl[pl.ds(v * 16, 16)] = jnp.zeros((16,), jnp.float32)
        lax.fori_loop(0, slc // 16, _z16, None)

        def _zb(v, _):
            mo_blk[pl.ds(v * 16, 16)] = jnp.zeros((16,), jnp.float32)
        lax.fori_loop(0, BLK // 16, _zb, None)

        def _zmw(b, _):
            e0 = ebase + b * BLK
            pltpu.sync_copy(mo_blk, ME.at[pl.ds(e0, BLK)])
            pltpu.sync_copy(mo_blk, WE.at[pl.ds(e0, BLK)])
        lax.fori_loop(0, nblk, _zmw, None)

        pltpu.sync_copy(zz_sl, acc_shared.at[pl.ds(nbase, slc)])
        plsc.subcore_barrier()

        def _cross_core_rendezvous():
            @pl.when(wid == 0)
            def _():
                pl.semaphore_signal(xsem, 1, core_index=1 - cid)
                pl.semaphore_wait(xsem, 1)
            plsc.subcore_barrier()

        def iter_body(t, _):
            @pl.when(t == 0)
            def _():
                pltpu.sync_copy(pi_hbm, beta_tab)

            @pl.when(t != 0)
            def _():
                pltpu.sync_copy(beta_hbm, beta_tab)

            def blk_body(b, _):
                e0 = ebase + b * BLK
                ins = [
                    pltpu.async_copy(srcE.at[pl.ds(e0, BLK)], src_blk, sem_in),
                    pltpu.async_copy(dstE.at[pl.ds(e0, BLK)], dst_blk, sem_in),
                    pltpu.async_copy(ME.at[pl.ds(e0, BLK)], m_blk, sem_in),
                    pltpu.async_copy(WE.at[pl.ds(e0, BLK)], w_blk, sem_in),
                ]
                for dsc in ins:
                    dsc.wait()

                @plsc.parallel_loop(0, BLK // 16, 1, unroll=8)
                def vec_body(v):
                    cs = pl.ds(v * 16, 16)
                    sv = src_blk[cs]
                    dv = dst_blk[cs]
                    mv = m_blk[cs]
                    wv = w_blk[cs]
                    bs = plsc.load_gather(beta_tab, [sv])
                    bd = plsc.load_gather(beta_tab, [dv])
                    mo_blk[cs] = _feval(lutb_v, luts_v, bs - wv)
                    wo_blk[cs] = _feval(lutb_v, luts_v, bd - mv)

                outs = [
                    pltpu.async_copy(mo_blk, ME.at[pl.ds(e0, BLK)], sem_out),
                    pltpu.async_copy(wo_blk, WE.at[pl.ds(e0, BLK)], sem_out),
                ]
                scat = pltpu.async_copy(wo_blk, acc_shared.at[src_blk],
                                        sem_scat, add=True)
                for dsc in outs:
                    dsc.wait()
                scat.wait()
            lax.fori_loop(0, nblk, blk_body, None)
            plsc.subcore_barrier()

            pltpu.sync_copy(acc_shared.at[pl.ds(nbase, slc)], acc_sl)

            @pl.when(cid == 0)
            def _():
                pltpu.sync_copy(acc_sl, acc0H.at[pl.ds(nbase, slc)])

            @pl.when(cid == 1)
            def _():
                pltpu.sync_copy(acc_sl, acc1H.at[pl.ds(nbase, slc)])

            pltpu.sync_copy(zz_sl, acc_shared.at[pl.ds(nbase, slc)])
            plsc.subcore_barrier()
            _cross_core_rendezvous()

            pltpu.sync_copy(acc0H.at[pl.ds(gbase, slg)], acc_sl.at[pl.ds(0, slg)])
            pltpu.sync_copy(acc1H.at[pl.ds(gbase, slg)], acc1_sl)

            def nb(v, _):
                vs = pl.ds(v * 16, 16)
                beta_sl[vs] = pi_sl[vs] + acc_sl[vs] + acc1_sl[vs]
            lax.fori_loop(0, slg // 16, nb, None)

            pltpu.sync_copy(beta_sl, beta_hbm.at[pl.ds(gbase, slg)])

            @pl.when(t == MAX_ITERS_K - 1)
            def _():
                def fin(v, _):
                    vs = pl.ds(v * 16, 16)
                    p0 = 1.0 / (1.0 + jnp.exp(-beta_sl[vs]))
                    prob_sl[vs] = p0
                lax.fori_loop(0, slg // 16, fin, None)
                pltpu.sync_copy(prob_sl, b0_hbm.at[pl.ds(gbase, slg)])

                def fin2(v, _):
                    vs = pl.ds(v * 16, 16)
                    prob_sl[vs] = 1.0 - prob_sl[vs]
                lax.fori_loop(0, slg // 16, fin2, None)
                pltpu.sync_copy(prob_sl, b1_hbm.at[pl.ds(gbase, slg)])

            plsc.subcore_barrier()
            _cross_core_rendezvous()
        lax.fori_loop(0, MAX_ITERS_K, iter_body, None)

    f32 = jnp.float32
    return pl.kernel(
        body,
        out_type=(
            jax.ShapeDtypeStruct((npad,), f32),
            jax.ShapeDtypeStruct((npad,), f32),
            jax.ShapeDtypeStruct((nnz_pad,), f32),
            jax.ShapeDtypeStruct((nnz_pad,), f32),
            jax.ShapeDtypeStruct((npad,), f32),
            jax.ShapeDtypeStruct((npad,), f32),
            jax.ShapeDtypeStruct((npad,), f32),
        ),
        mesh=mesh,
        scratch_types=[
            pltpu.VMEM((npad,), f32),
            pltpu.VMEM((LN,), f32),
            pltpu.VMEM((LN,), f32),
            pltpu.VMEM((BLK,), jnp.int32),
            pltpu.VMEM((BLK,), jnp.int32),
            pltpu.VMEM((BLK,), f32),
            pltpu.VMEM((BLK,), f32),
            pltpu.VMEM((BLK,), f32),
            pltpu.VMEM((BLK,), f32),
            pltpu.VMEM((npad // (NC * NT),), f32),
            pltpu.VMEM((npad // NT,), f32),
            pltpu.VMEM((npad // (NC * NT),), f32),
            pltpu.VMEM((npad // (NC * NT),), f32),
            pltpu.VMEM((npad // NT,), f32),
            pltpu.VMEM((npad // (NC * NT),), f32),
            pltpu.VMEM_SHARED((npad,), f32),
            pltpu.SemaphoreType.DMA,
            pltpu.SemaphoreType.DMA,
            pltpu.SemaphoreType.DMA,
            pltpu.SemaphoreType.REGULAR,
        ],
        compiler_params=pltpu.CompilerParams(needs_layout_passes=False),
        interpret=interpret,
    )


def _prepare(priors, potential, src_nodes, dst_nodes):
    n = priors.shape[0]
    nnz = src_nodes.shape[0]
    nblk = -(-nnz // (NC * NT * BLK))
    nnz_pad = NC * NT * nblk * BLK
    npad = -(-(n + PAD_SPREAD) // (NC * NT * 16)) * (NC * NT * 16)

    i32 = jnp.int32
    pad_idx = (n + (jnp.arange(nnz_pad - nnz, dtype=i32) % PAD_SPREAD))
    src_p = jnp.concatenate([src_nodes.astype(i32), pad_idx])
    dst_p = jnp.concatenate([dst_nodes.astype(i32), pad_idx])

    logpr = jnp.log(priors)
    pi = logpr[:, 0] - logpr[:, 1]
    pi_pad = jnp.concatenate(
        [pi, jnp.zeros((npad - n,), jnp.float32)])

    k = potential[0, 0] / potential[0, 1]
    grid = jnp.arange(LN + 1, dtype=jnp.float32) * jnp.float32(1.0 / LUT_SCALE)
    tt = jnp.exp(-grid)
    fv = jnp.log((k + tt) / (1.0 + k * tt))
    lutb = fv[:LN]
    luts = fv[1:] - fv[:-1]
    return npad, nnz_pad, nblk, src_p, dst_p, pi_pad, lutb, luts


def kernel(priors, potential, src_nodes, dst_nodes, rev_edges):
    del rev_edges
    n = priors.shape[0]
    npad, nnz_pad, nblk, src_p, dst_p, pi_pad, lutb, luts = _prepare(
        priors, potential, src_nodes, dst_nodes)
    fn = _make_bp(npad, nnz_pad, nblk)
    b0, b1, _m, _w, _beta, _a0, _a1 = fn(src_p, dst_p, pi_pad, lutb, luts)
    return jnp.stack([b0[:n], b1[:n]], axis=1)


# speedup vs baseline: 136.2378x; 1.3211x over previous
import jax
import jax.numpy as jnp
from jax import lax
from jax.experimental import pallas as pl
from jax.experimental.pallas import tpu as pltpu
import jax.experimental.pallas.tpu_sc as plsc

MAX_ITERS_K = 10
NC = 2
NT = 16
BLK = 2048
LN = 2048
DMAX = 16.0
LUT_SCALE = LN / DMAX
PAD_SPREAD = 1024


def _feval(lutb_v, luts_v, delta):
    a = jnp.abs(delta)
    scaled = jnp.minimum(a * jnp.float32(LUT_SCALE), jnp.float32(LN) - 0.5)
    idx = scaled.astype(jnp.int32)
    frac = scaled - idx.astype(jnp.float32)
    g = plsc.load_gather(lutb_v, [idx]) + frac * plsc.load_gather(luts_v, [idx])
    return jnp.where(delta < 0.0, -g, g)


def _make_bp(npad, nnz_pad, nblk, interpret=False):
    ept = nblk * BLK
    slc = npad // NT
    slg = npad // (NC * NT)
    mesh = plsc.VectorSubcoreMesh(
        core_axis_name="c", subcore_axis_name="s",
        num_cores=NC, num_subcores=NT)

    def body(srcE, dstE, pi_hbm, lutb_hbm, luts_hbm,
             b0_hbm, b1_hbm, beta_hbm, acc0H, acc1H,
             beta_tab, lutb_v, luts_v,
             src_a, src_b, dst_a, dst_b, m_res, w_res,
             pi_sl, acc_sl, acc1_sl, beta_sl, prob_sl, acc_shared,
             sem_in, sem_beta, sem_scat, xsem):
        cid = lax.axis_index("c")
        wid = lax.axis_index("s")
        gid = cid * NT + wid
        ebase = gid * ept
        nbase = wid * slc
        gbase = gid * slg
        src_bufs = [src_a, src_b]
        dst_bufs = [dst_a, dst_b]

        pltpu.sync_copy(lutb_hbm, lutb_v)
        pltpu.sync_copy(luts_hbm, luts_v)
        pltpu.sync_copy(pi_hbm.at[pl.ds(gbase, slg)], pi_sl)

        @plsc.parallel_loop(0, ept // 16, 1, unroll=8)
        def _zmw(v):
            zs = pl.ds(v * 16, 16)
            m_res[zs] = jnp.zeros((16,), jnp.float32)
            w_res[zs] = jnp.zeros((16,), jnp.float32)

        def _zp(v, _):
            prob_sl[pl.ds(v * 16, 16)] = jnp.zeros((16,), jnp.float32)
        lax.fori_loop(0, slg // 16, _zp, None)

        def _zero_acc_slice():
            pltpu.sync_copy(prob_sl, acc_shared.at[pl.ds(nbase, slg)])
            pltpu.sync_copy(prob_sl, acc_shared.at[pl.ds(nbase + slg, slg)])

        _zero_acc_slice()
        pltpu.sync_copy(pi_sl, beta_hbm.at[pl.ds(gbase, slg)])
        plsc.subcore_barrier()

        def _cross_core_rendezvous():
            @pl.when(wid == 0)
            def _():
                pl.semaphore_signal(xsem, 1, core_index=1 - cid)
                pl.semaphore_wait(xsem, 1)
            plsc.subcore_barrier()

        _cross_core_rendezvous()

        def _fire_in(b):
            p = b % 2
            return (
                pltpu.async_copy(srcE.at[pl.ds(ebase + b * BLK, BLK)],
                                 src_bufs[p], sem_in),
                pltpu.async_copy(dstE.at[pl.ds(ebase + b * BLK, BLK)],
                                 dst_bufs[p], sem_in),
            )

        def iter_body(t, _):
            beta_d = pltpu.async_copy(beta_hbm, beta_tab, sem_beta)
            ins = {0: _fire_in(0)}
            scats = {}
            for b in range(nblk):
                p = b % 2
                for dsc in ins.pop(b):
                    dsc.wait()
                if b >= 1:
                    scats.pop(b - 1).wait()
                if b + 1 < nblk:
                    ins[b + 1] = _fire_in(b + 1)
                if b == 0:
                    beta_d.wait()
                base = b * BLK

                @plsc.parallel_loop(0, BLK // 16, 1, unroll=8)
                def vec_body(v):
                    cs = pl.ds(v * 16, 16)
                    ms = pl.ds(base + v * 16, 16)
                    sv = src_bufs[p][cs]
                    dv = dst_bufs[p][cs]
                    mv = m_res[ms]
                    wv = w_res[ms]
                    bs = plsc.load_gather(beta_tab, [sv])
                    bd = plsc.load_gather(beta_tab, [dv])
                    m_res[ms] = _feval(lutb_v, luts_v, bs - wv)
                    w_res[ms] = _feval(lutb_v, luts_v, bd - mv)

                scats[b] = pltpu.async_copy(
                    w_res.at[pl.ds(base, BLK)],
                    acc_shared.at[src_bufs[p]], sem_scat, add=True)
            scats.pop(nblk - 1).wait()
            plsc.subcore_barrier()

            pltpu.sync_copy(acc_shared.at[pl.ds(nbase, slc)], acc_sl)

            @pl.when(cid == 0)
            def _():
                pltpu.sync_copy(acc_sl, acc0H.at[pl.ds(nbase, slc)])

            @pl.when(cid == 1)
            def _():
                pltpu.sync_copy(acc_sl, acc1H.at[pl.ds(nbase, slc)])

            _zero_acc_slice()
            plsc.subcore_barrier()
            _cross_core_rendezvous()

            pltpu.sync_copy(acc0H.at[pl.ds(gbase, slg)],
                            acc_sl.at[pl.ds(0, slg)])
            pltpu.sync_copy(acc1H.at[pl.ds(gbase, slg)], acc1_sl)

            def nb(v, _):
                vs = pl.ds(v * 16, 16)
                beta_sl[vs] = pi_sl[vs] + acc_sl[vs] + acc1_sl[vs]
            lax.fori_loop(0, slg // 16, nb, None)

            pltpu.sync_copy(beta_sl, beta_hbm.at[pl.ds(gbase, slg)])

            @pl.when(t == MAX_ITERS_K - 1)
            def _():
                def fin(v, _):
                    vs = pl.ds(v * 16, 16)
                    p0 = 1.0 / (1.0 + jnp.exp(-beta_sl[vs]))
                    prob_sl[vs] = p0
                lax.fori_loop(0, slg // 16, fin, None)
                pltpu.sync_copy(prob_sl, b0_hbm.at[pl.ds(gbase, slg)])

                def fin2(v, _):
                    vs = pl.ds(v * 16, 16)
                    prob_sl[vs] = 1.0 - prob_sl[vs]
                lax.fori_loop(0, slg // 16, fin2, None)
                pltpu.sync_copy(prob_sl, b1_hbm.at[pl.ds(gbase, slg)])

            plsc.subcore_barrier()
            _cross_core_rendezvous()
        lax.fori_loop(0, MAX_ITERS_K, iter_body, None)

    f32 = jnp.float32
    return pl.kernel(
        body,
        out_type=(
            jax.ShapeDtypeStruct((npad,), f32),
            jax.ShapeDtypeStruct((npad,), f32),
            jax.ShapeDtypeStruct((npad,), f32),
            jax.ShapeDtypeStruct((npad,), f32),
            jax.ShapeDtypeStruct((npad,), f32),
        ),
        mesh=mesh,
        scratch_types=[
            pltpu.VMEM((npad,), f32),
            pltpu.VMEM((LN,), f32),
            pltpu.VMEM((LN,), f32),
            pltpu.VMEM((BLK,), jnp.int32),
            pltpu.VMEM((BLK,), jnp.int32),
            pltpu.VMEM((BLK,), jnp.int32),
            pltpu.VMEM((BLK,), jnp.int32),
            pltpu.VMEM((ept,), f32),
            pltpu.VMEM((ept,), f32),
            pltpu.VMEM((npad // (NC * NT),), f32),
            pltpu.VMEM((npad // NT,), f32),
            pltpu.VMEM((npad // (NC * NT),), f32),
            pltpu.VMEM((npad // (NC * NT),), f32),
            pltpu.VMEM((npad // (NC * NT),), f32),
            pltpu.VMEM_SHARED((npad,), f32),
            pltpu.SemaphoreType.DMA,
            pltpu.SemaphoreType.DMA,
            pltpu.SemaphoreType.DMA,
            pltpu.SemaphoreType.REGULAR,
        ],
        compiler_params=pltpu.CompilerParams(needs_layout_passes=False),
        interpret=interpret,
    )


def _prepare(priors, potential, src_nodes, dst_nodes):
    n = priors.shape[0]
    nnz = src_nodes.shape[0]
    nblk = -(-nnz // (NC * NT * BLK))
    nnz_pad = NC * NT * nblk * BLK
    npad = -(-(n + PAD_SPREAD) // (NC * NT * 16)) * (NC * NT * 16)

    i32 = jnp.int32
    pad_idx = (n + (jnp.arange(nnz_pad - nnz, dtype=i32) % PAD_SPREAD))
    src_p = jnp.concatenate([src_nodes.astype(i32), pad_idx])
    dst_p = jnp.concatenate([dst_nodes.astype(i32), pad_idx])

    logpr = jnp.log(priors)
    pi = logpr[:, 0] - logpr[:, 1]
    pi_pad = jnp.concatenate(
        [pi, jnp.zeros((npad - n,), jnp.float32)])

    k = potential[0, 0] / potential[0, 1]
    grid = jnp.arange(LN + 1, dtype=jnp.float32) * jnp.float32(1.0 / LUT_SCALE)
    tt = jnp.exp(-grid)
    fv = jnp.log((k + tt) / (1.0 + k * tt))
    lutb = fv[:LN]
    luts = fv[1:] - fv[:-1]
    return npad, nnz_pad, nblk, src_p, dst_p, pi_pad, lutb, luts


def kernel(priors, potential, src_nodes, dst_nodes, rev_edges):
    del rev_edges
    n = priors.shape[0]
    npad, nnz_pad, nblk, src_p, dst_p, pi_pad, lutb, luts = _prepare(
        priors, potential, src_nodes, dst_nodes)
    fn = _make_bp(npad, nnz_pad, nblk)
    b0, b1, _beta, _a0, _a1 = fn(src_p, dst_p, pi_pad, lutb, luts)
    return jnp.stack([b0[:n], b1[:n]], axis=1)


# speedup vs baseline: 151.4007x; 1.1113x over previous
import jax
import jax.numpy as jnp
from jax import lax
from jax.experimental import pallas as pl
from jax.experimental.pallas import tpu as pltpu
import jax.experimental.pallas.tpu_sc as plsc

MAX_ITERS_K = 10
NC = 2
NT = 16
BLK = 2048
LN = 4096
DMAX = 16.0
LUT_SCALE = LN / DMAX
PAD_SPREAD = 1024


def _feval(lut_v, delta):
    a = jnp.abs(delta)
    scaled = jnp.minimum(a * jnp.float32(LUT_SCALE), jnp.float32(LN) - 0.5)
    idx = scaled.astype(jnp.int32)
    g = plsc.load_gather(lut_v, [idx])
    return jnp.where(delta < 0.0, -g, g)


def _make_bp(npad, nnz_pad, nblk, interpret=False):
    ept = nblk * BLK
    slc = npad // NT
    slg = npad // (NC * NT)
    mesh = plsc.VectorSubcoreMesh(
        core_axis_name="c", subcore_axis_name="s",
        num_cores=NC, num_subcores=NT)

    def body(pairE, pi_hbm, lut_hbm,
             b0_hbm, b1_hbm, beta_hbm, acc0H, acc1H,
             beta_tab, lut_v,
             pk_a, pk_b, sc_a, sc_b, m_res, w_res,
             pi_sl, acc_sl, acc1_sl, beta_sl, prob_sl, acc_shared,
             sem_in, sem_beta, sem_scat, xsem):
        cid = lax.axis_index("c")
        wid = lax.axis_index("s")
        gid = cid * NT + wid
        ebase = gid * ept
        nbase = wid * slc
        gbase = gid * slg
        pk_bufs = [pk_a, pk_b]
        sc_bufs = [sc_a, sc_b]

        pltpu.sync_copy(lut_hbm, lut_v)
        pltpu.sync_copy(pi_hbm.at[pl.ds(gbase, slg)], pi_sl)

        @plsc.parallel_loop(0, ept // 16, 1, unroll=8)
        def _zmw(v):
            zs = pl.ds(v * 16, 16)
            m_res[zs] = jnp.zeros((16,), jnp.float32)
            w_res[zs] = jnp.zeros((16,), jnp.float32)

        def _zp(v, _):
            prob_sl[pl.ds(v * 16, 16)] = jnp.zeros((16,), jnp.float32)
        lax.fori_loop(0, slg // 16, _zp, None)

        def _zero_acc_slice():
            pltpu.sync_copy(prob_sl, acc_shared.at[pl.ds(nbase, slg)])
            pltpu.sync_copy(prob_sl, acc_shared.at[pl.ds(nbase + slg, slg)])

        _zero_acc_slice()
        pltpu.sync_copy(pi_sl, beta_hbm.at[pl.ds(gbase, slg)])
        plsc.subcore_barrier()

        def _cross_core_rendezvous():
            @pl.when(wid == 0)
            def _():
                pl.semaphore_signal(xsem, 1, core_index=1 - cid)
                pl.semaphore_wait(xsem, 1)
            plsc.subcore_barrier()

        _cross_core_rendezvous()

        def _fire_in(b):
            p = b % 2
            return (
                pltpu.async_copy(pairE.at[pl.ds(ebase + b * BLK, BLK)],
                                 pk_bufs[p], sem_in),
            )

        def iter_body(t, _):
            beta_d = pltpu.async_copy(beta_hbm, beta_tab, sem_beta)
            ins = {0: _fire_in(0)}
            scats = {}
            for b in range(nblk):
                p = b % 2
                for dsc in ins.pop(b):
                    dsc.wait()
                if b >= 1:
                    scats.pop(b - 1).wait()
                if b + 1 < nblk:
                    ins[b + 1] = _fire_in(b + 1)
                if b == 0:
                    beta_d.wait()
                base = b * BLK

                @plsc.parallel_loop(0, BLK // 16, 1, unroll=8)
                def vec_body(v):
                    cs = pl.ds(v * 16, 16)
                    ms = pl.ds(base + v * 16, 16)
                    pk = pk_bufs[p][cs]
                    sv = (pk & jnp.uint32(0xFFFF)).astype(jnp.int32)
                    dv = (pk >> jnp.uint32(16)).astype(jnp.int32)
                    sc_bufs[p][cs] = sv
                    mv = m_res[ms]
                    wv = w_res[ms]
                    bs = plsc.load_gather(beta_tab, [sv])
                    bd = plsc.load_gather(beta_tab, [dv])
                    m_res[ms] = _feval(lut_v, bs - wv)
                    w_res[ms] = _feval(lut_v, bd - mv)

                scats[b] = pltpu.async_copy(
                    w_res.at[pl.ds(base, BLK)],
                    acc_shared.at[sc_bufs[p]], sem_scat, add=True)
            scats.pop(nblk - 1).wait()
            plsc.subcore_barrier()

            pltpu.sync_copy(acc_shared.at[pl.ds(nbase, slc)], acc_sl)

            @pl.when(cid == 0)
            def _():
                pltpu.sync_copy(acc_sl, acc0H.at[pl.ds(nbase, slc)])

            @pl.when(cid == 1)
            def _():
                pltpu.sync_copy(acc_sl, acc1H.at[pl.ds(nbase, slc)])

            _zero_acc_slice()
            plsc.subcore_barrier()
            _cross_core_rendezvous()

            pltpu.sync_copy(acc0H.at[pl.ds(gbase, slg)],
                            acc_sl.at[pl.ds(0, slg)])
            pltpu.sync_copy(acc1H.at[pl.ds(gbase, slg)], acc1_sl)

            def nb(v, _):
                vs = pl.ds(v * 16, 16)
                beta_sl[vs] = pi_sl[vs] + acc_sl[vs] + acc1_sl[vs]
            lax.fori_loop(0, slg // 16, nb, None)

            pltpu.sync_copy(beta_sl, beta_hbm.at[pl.ds(gbase, slg)])

            @pl.when(t == MAX_ITERS_K - 1)
            def _():
                def fin(v, _):
                    vs = pl.ds(v * 16, 16)
                    p0 = 1.0 / (1.0 + jnp.exp(-beta_sl[vs]))
                    prob_sl[vs] = p0
                lax.fori_loop(0, slg // 16, fin, None)
                pltpu.sync_copy(prob_sl, b0_hbm.at[pl.ds(gbase, slg)])

                def fin2(v, _):
                    vs = pl.ds(v * 16, 16)
                    prob_sl[vs] = 1.0 - prob_sl[vs]
                lax.fori_loop(0, slg // 16, fin2, None)
                pltpu.sync_copy(prob_sl, b1_hbm.at[pl.ds(gbase, slg)])

            plsc.subcore_barrier()
            _cross_core_rendezvous()
        lax.fori_loop(0, MAX_ITERS_K, iter_body, None)

    f32 = jnp.float32
    return pl.kernel(
        body,
        out_type=(
            jax.ShapeDtypeStruct((npad,), f32),
            jax.ShapeDtypeStruct((npad,), f32),
            jax.ShapeDtypeStruct((npad,), f32),
            jax.ShapeDtypeStruct((npad,), f32),
            jax.ShapeDtypeStruct((npad,), f32),
        ),
        mesh=mesh,
        scratch_types=[
            pltpu.VMEM((npad,), f32),
            pltpu.VMEM((LN,), f32),
            pltpu.VMEM((BLK,), jnp.uint32),
            pltpu.VMEM((BLK,), jnp.uint32),
            pltpu.VMEM((BLK,), jnp.int32),
            pltpu.VMEM((BLK,), jnp.int32),
            pltpu.VMEM((ept,), f32),
            pltpu.VMEM((ept,), f32),
            pltpu.VMEM((npad // (NC * NT),), f32),
            pltpu.VMEM((npad // NT,), f32),
            pltpu.VMEM((npad // (NC * NT),), f32),
            pltpu.VMEM((npad // (NC * NT),), f32),
            pltpu.VMEM((npad // (NC * NT),), f32),
            pltpu.VMEM_SHARED((npad,), f32),
            pltpu.SemaphoreType.DMA,
            pltpu.SemaphoreType.DMA,
            pltpu.SemaphoreType.DMA,
            pltpu.SemaphoreType.REGULAR,
        ],
        compiler_params=pltpu.CompilerParams(needs_layout_passes=False),
        interpret=interpret,
    )


def _prepare(priors, potential, src_nodes, dst_nodes):
    n = priors.shape[0]
    nnz = src_nodes.shape[0]
    nblk = -(-nnz // (NC * NT * BLK))
    nnz_pad = NC * NT * nblk * BLK
    npad = -(-(n + PAD_SPREAD) // (NC * NT * 16)) * (NC * NT * 16)
    assert npad <= 65536, "packed u32 index layout needs node ids in 16 bits"

    u32 = jnp.uint32
    pad_idx = (n + (jnp.arange(nnz_pad - nnz, dtype=jnp.int32) % PAD_SPREAD))
    src_p = jnp.concatenate([src_nodes.astype(jnp.int32), pad_idx]).astype(u32)
    dst_p = jnp.concatenate([dst_nodes.astype(jnp.int32), pad_idx]).astype(u32)
    pair_p = src_p | (dst_p << u32(16))

    logpr = jnp.log(priors)
    pi = logpr[:, 0] - logpr[:, 1]
    pi_pad = jnp.concatenate(
        [pi, jnp.zeros((npad - n,), jnp.float32)])

    k = potential[0, 0] / potential[0, 1]
    mid = (jnp.arange(LN, dtype=jnp.float32) + 0.5) * jnp.float32(1.0 / LUT_SCALE)
    tt = jnp.exp(-mid)
    lut = jnp.log((k + tt) / (1.0 + k * tt))
    return npad, nnz_pad, nblk, pair_p, pi_pad, lut


def kernel(priors, potential, src_nodes, dst_nodes, rev_edges):
    del rev_edges
    n = priors.shape[0]
    npad, nnz_pad, nblk, pair_p, pi_pad, lut = _prepare(
        priors, potential, src_nodes, dst_nodes)
    fn = _make_bp(npad, nnz_pad, nblk)
    b0, b1, _beta, _a0, _a1 = fn(pair_p, pi_pad, lut)
    return jnp.stack([b0[:n], b1[:n]], axis=1)


# speedup vs baseline: 187.3437x; 1.2374x over previous
import jax
import jax.numpy as jnp
from jax import lax
from jax.experimental import pallas as pl
from jax.experimental.pallas import tpu as pltpu
import jax.experimental.pallas.tpu_sc as plsc

MAX_ITERS_K = 10
NC = 2
NT = 16
BLK = 2048
LN = 2048
DMAX = 16.0
LUT_SCALE = LN / DMAX
PAD_SPREAD = 1024


def _feval(lut_v, delta):
    a = jnp.abs(delta)
    scaled = jnp.minimum(a * jnp.float32(LUT_SCALE), jnp.float32(LN) - 0.5)
    idx = scaled.astype(jnp.int32)
    g = plsc.load_gather(lut_v, [idx])
    return jnp.where(delta < 0.0, -g, g)


def _make_bp(npad, nnz_pad, nblk, interpret=False):
    ept = nblk * BLK
    slc = npad // NT
    slg = npad // (NC * NT)
    mesh = plsc.VectorSubcoreMesh(
        core_axis_name="c", subcore_axis_name="s",
        num_cores=NC, num_subcores=NT)

    def body(pairE, pi_hbm, lut_hbm,
             b0_hbm, b1_hbm, beta_hbm, acc0H, acc1H,
             beta_tab, lut_v,
             pk_a, pk_b, pk_c, sc_a, sc_b, sc_c, m_res, w_res,
             pi_sl, acc_sl, acc1_sl, beta_sl, prob_sl, acc_shared,
             sem_in, sem_beta, sem_scat, xsem):
        cid = lax.axis_index("c")
        wid = lax.axis_index("s")
        gid = cid * NT + wid
        ebase = gid * ept
        nbase = wid * slc
        gbase = gid * slg
        pk_bufs = [pk_a, pk_b, pk_c]
        sc_bufs = [sc_a, sc_b, sc_c]

        pltpu.sync_copy(lut_hbm, lut_v)
        pltpu.sync_copy(pi_hbm.at[pl.ds(gbase, slg)], pi_sl)

        @plsc.parallel_loop(0, ept // 16, 1, unroll=8)
        def _zmw(v):
            zs = pl.ds(v * 16, 16)
            m_res[zs] = jnp.zeros((16,), jnp.float32)
            w_res[zs] = jnp.zeros((16,), jnp.float32)

        def _zp(v, _):
            prob_sl[pl.ds(v * 16, 16)] = jnp.zeros((16,), jnp.float32)
        lax.fori_loop(0, slg // 16, _zp, None)

        def _zero_acc_slice():
            pltpu.sync_copy(prob_sl, acc_shared.at[pl.ds(nbase, slg)])
            pltpu.sync_copy(prob_sl, acc_shared.at[pl.ds(nbase + slg, slg)])

        _zero_acc_slice()
        pltpu.sync_copy(pi_sl, beta_hbm.at[pl.ds(gbase, slg)])
        plsc.subcore_barrier()

        def _cross_core_rendezvous():
            @pl.when(wid == 0)
            def _():
                pl.semaphore_signal(xsem, 1, core_index=1 - cid)
                pl.semaphore_wait(xsem, 1)
            plsc.subcore_barrier()

        _cross_core_rendezvous()

        def _fire_in(b):
            p = b % 3
            return (
                pltpu.async_copy(pairE.at[pl.ds(ebase + b * BLK, BLK)],
                                 pk_bufs[p], sem_in),
            )

        def iter_body(t, _):
            beta_d = pltpu.async_copy(beta_hbm, beta_tab, sem_beta)
            ins = {0: _fire_in(0)}
            scats = {}
            for b in range(nblk):
                p = b % 3
                for dsc in ins.pop(b):
                    dsc.wait()
                if b - 2 in scats:
                    scats.pop(b - 2).wait()
                if b + 1 < nblk:
                    ins[b + 1] = _fire_in(b + 1)
                if b == 0:
                    beta_d.wait()
                base = b * BLK

                @plsc.parallel_loop(0, BLK // 16, 1, unroll=8)
                def vec_body(v):
                    cs = pl.ds(v * 16, 16)
                    ms = pl.ds(base + v * 16, 16)
                    pk = pk_bufs[p][cs]
                    sv = (pk & jnp.uint32(0xFFFF)).astype(jnp.int32)
                    dv = (pk >> jnp.uint32(16)).astype(jnp.int32)
                    sc_bufs[p][cs] = sv
                    mv = m_res[ms]
                    wv = w_res[ms]
                    bs = plsc.load_gather(beta_tab, [sv])
                    bd = plsc.load_gather(beta_tab, [dv])
                    m_res[ms] = _feval(lut_v, bs - wv)
                    w_res[ms] = _feval(lut_v, bd - mv)

                scats[b] = pltpu.async_copy(
                    w_res.at[pl.ds(base, BLK)],
                    acc_shared.at[sc_bufs[p]], sem_scat, add=True)
            for b_left in sorted(scats):
                scats.pop(b_left).wait()
            plsc.subcore_barrier()

            for h in range(slc // slg):
                hb = nbase + h * slg
                pltpu.sync_copy(acc_shared.at[pl.ds(hb, slg)], acc_sl)

                @pl.when(cid == 0)
                def _():
                    pltpu.sync_copy(acc_sl, acc0H.at[pl.ds(hb, slg)])

                @pl.when(cid == 1)
                def _():
                    pltpu.sync_copy(acc_sl, acc1H.at[pl.ds(hb, slg)])

            _zero_acc_slice()
            plsc.subcore_barrier()
            _cross_core_rendezvous()

            pltpu.sync_copy(acc0H.at[pl.ds(gbase, slg)], acc_sl)
            pltpu.sync_copy(acc1H.at[pl.ds(gbase, slg)], acc1_sl)

            def nb(v, _):
                vs = pl.ds(v * 16, 16)
                beta_sl[vs] = pi_sl[vs] + acc_sl[vs] + acc1_sl[vs]
            lax.fori_loop(0, slg // 16, nb, None)

            pltpu.sync_copy(beta_sl, beta_hbm.at[pl.ds(gbase, slg)])

            @pl.when(t == MAX_ITERS_K - 1)
            def _():
                def fin(v, _):
                    vs = pl.ds(v * 16, 16)
                    p0 = 1.0 / (1.0 + jnp.exp(-beta_sl[vs]))
                    prob_sl[vs] = p0
                lax.fori_loop(0, slg // 16, fin, None)
                pltpu.sync_copy(prob_sl, b0_hbm.at[pl.ds(gbase, slg)])

                def fin2(v, _):
                    vs = pl.ds(v * 16, 16)
                    prob_sl[vs] = 1.0 - prob_sl[vs]
                lax.fori_loop(0, slg // 16, fin2, None)
                pltpu.sync_copy(prob_sl, b1_hbm.at[pl.ds(gbase, slg)])

            plsc.subcore_barrier()
            _cross_core_rendezvous()
        lax.fori_loop(0, MAX_ITERS_K, iter_body, None)

    f32 = jnp.float32
    return pl.kernel(
        body,
        out_type=(
            jax.ShapeDtypeStruct((npad,), f32),
            jax.ShapeDtypeStruct((npad,), f32),
            jax.ShapeDtypeStruct((npad,), f32),
            jax.ShapeDtypeStruct((npad,), f32),
            jax.ShapeDtypeStruct((npad,), f32),
        ),
        mesh=mesh,
        scratch_types=[
            pltpu.VMEM((npad,), f32),
            pltpu.VMEM((LN,), f32),
            pltpu.VMEM((BLK,), jnp.uint32),
            pltpu.VMEM((BLK,), jnp.uint32),
            pltpu.VMEM((BLK,), jnp.uint32),
            pltpu.VMEM((BLK,), jnp.int32),
            pltpu.VMEM((BLK,), jnp.int32),
            pltpu.VMEM((BLK,), jnp.int32),
            pltpu.VMEM((ept,), f32),
            pltpu.VMEM((ept,), f32),
            pltpu.VMEM((npad // (NC * NT),), f32),
            pltpu.VMEM((npad // (NC * NT),), f32),
            pltpu.VMEM((npad // (NC * NT),), f32),
            pltpu.VMEM((npad // (NC * NT),), f32),
            pltpu.VMEM((npad // (NC * NT),), f32),
            pltpu.VMEM_SHARED((npad,), f32),
            pltpu.SemaphoreType.DMA,
            pltpu.SemaphoreType.DMA,
            pltpu.SemaphoreType.DMA,
            pltpu.SemaphoreType.REGULAR,
        ],
        compiler_params=pltpu.CompilerParams(needs_layout_passes=False),
        interpret=interpret,
    )


def _prepare(priors, potential, src_nodes, dst_nodes):
    n = priors.shape[0]
    nnz = src_nodes.shape[0]
    nblk = -(-nnz // (NC * NT * BLK))
    nnz_pad = NC * NT * nblk * BLK
    npad = -(-(n + PAD_SPREAD) // (NC * NT * 16)) * (NC * NT * 16)
    assert npad <= 65536, "packed u32 index layout needs node ids in 16 bits"

    u32 = jnp.uint32
    pad_idx = (n + (jnp.arange(nnz_pad - nnz, dtype=jnp.int32) % PAD_SPREAD))
    src_p = jnp.concatenate([src_nodes.astype(jnp.int32), pad_idx]).astype(u32)
    dst_p = jnp.concatenate([dst_nodes.astype(jnp.int32), pad_idx]).astype(u32)
    pair_p = src_p | (dst_p << u32(16))

    logpr = jnp.log(priors)
    pi = logpr[:, 0] - logpr[:, 1]
    pi_pad = jnp.concatenate(
        [pi, jnp.zeros((npad - n,), jnp.float32)])

    k = potential[0, 0] / potential[0, 1]
    mid = (jnp.arange(LN, dtype=jnp.float32) + 0.5) * jnp.float32(1.0 / LUT_SCALE)
    tt = jnp.exp(-mid)
    lut = jnp.log((k + tt) / (1.0 + k * tt))
    return npad, nnz_pad, nblk, pair_p, pi_pad, lut


def kernel(priors, potential, src_nodes, dst_nodes, rev_edges):
    del rev_edges
    n = priors.shape[0]
    npad, nnz_pad, nblk, pair_p, pi_pad, lut = _prepare(
        priors, potential, src_nodes, dst_nodes)
    fn = _make_bp(npad, nnz_pad, nblk)
    b0, b1, _beta, _a0, _a1 = fn(pair_p, pi_pad, lut)
    return jnp.stack([b0[:n], b1[:n]], axis=1)
